# baseline (speedup 1.0000x reference)
import jax
import jax.numpy as jnp
from jax import lax
from jax.experimental import pallas as pl
from jax.experimental.pallas import tpu as pltpu
from jax.experimental.pallas import tpu_sc as plsc

B, H, W, C = 8, 224, 224, 96
NBINS = 256
NB_INT = NBINS + 1
TILES_PER_B = 4
C_PER_TILE = C // TILES_PER_B
CPASS = 8
NPASS = C_PER_TILE // CPASS
HC = 16
NCHUNK = H // HC
LANES = 16
WK = W // LANES
CSTRIDE = 258 * LANES
H16 = CPASS * CSTRIDE
NG = NBINS // LANES
SROW = 17
SG = LANES * SROW


def _body(x_hbm, out_hbm, h16_v, stg_v, buf0, buf1, outb_v, sem0, sem1):
    cid = lax.axis_index("c")
    sid = lax.axis_index("s")
    b = cid * TILES_PER_B + sid // TILES_PER_B
    q = sid % TILES_PER_B
    c0 = q * C_PER_TILE

    zeros = jnp.zeros((LANES,), jnp.float32)
    ones = jnp.ones((LANES,), jnp.float32)
    iota = lax.iota(jnp.int32, LANES)
    iota17 = iota * SROW

    @plsc.parallel_loop(0, H16 // LANES)
    def _(i):
        h16_v[pl.ds(i * LANES, LANES)] = zeros

    cbs = [iota + ci * CSTRIDE for ci in range(CPASS)]

    def chunk_compute(buf):
        @plsc.parallel_loop(0, HC)
        def _(h):
            for ci in range(CPASS):
                cb = cbs[ci]
                for k in range(WK):
                    v = buf[h, ci, pl.ds(k * LANES, LANES)]
                    t = (v * jnp.float32(NB_INT)).astype(jnp.int32)
                    plsc.addupdate_scatter(h16_v, [(t << 4) + cb], ones)

    def start(p, n, buf, sem):
        pltpu.async_copy(
            x_hbm.at[b, pl.ds(n * HC, HC), pl.ds(c0 + p * CPASS, CPASS), :],
            buf, sem)

    def wait(buf, sem):
        pltpu.make_async_copy(
            x_hbm.at[0, pl.ds(0, HC), pl.ds(0, CPASS), :], buf, sem).wait()

    def pass_body(p, _):
        start(p, 0, buf0, sem0)

        def pair_body(k, _):
            start(p, 2 * k + 1, buf1, sem1)
            wait(buf0, sem0)
            chunk_compute(buf0)

            @pl.when(k < NCHUNK // 2 - 1)
            def _():
                start(p, 2 * k + 2, buf0, sem0)

            wait(buf1, sem1)
            chunk_compute(buf1)
            return 0

        lax.fori_loop(0, NCHUNK // 2, pair_body, 0)

        def fold_body(ci, _):
            cb2 = ci * CSTRIDE
            row = p * CPASS + ci

            idxs = [iota17 + j2 for j2 in range(LANES)]

            @plsc.parallel_loop(0, NG, carry=(zeros,))
            def den_acc(g, dcar):
                sb = g * SG
                stg = stg_v.at[pl.ds(sb, SG)]
                for j2 in range(LANES):
                    o = cb2 + g * (LANES * LANES) + (j2 + 1) * LANES
                    v = h16_v[pl.ds(o, LANES)]
                    plsc.store_scatter(stg, [idxs[j2]], v)
                    h16_v[pl.ds(o, LANES)] = zeros
                acc = stg_v[pl.ds(sb, LANES)]
                for ll in range(1, LANES):
                    acc = acc + stg_v[pl.ds(sb + ll * SROW, LANES)]
                outb_v[row, pl.ds(g * LANES, LANES)] = acc
                return (dcar[0] + acc,)

            den = jnp.maximum(jnp.sum(den_acc[0]), jnp.float32(1e-7))
            inv = jnp.float32(1.0) / lax.broadcast_in_dim(den, (LANES,), ())
            for g in range(NG):
                outb_v[row, pl.ds(g * LANES, LANES)] = (
                    outb_v[row, pl.ds(g * LANES, LANES)] * inv)
            h16_v[pl.ds(cb2, LANES)] = zeros
            return 0

        lax.fori_loop(0, CPASS, fold_body, 0)
        return 0

    lax.fori_loop(0, NPASS, pass_body, 0)

    pltpu.sync_copy(outb_v, out_hbm.at[b, pl.ds(c0, C_PER_TILE), :])


@jax.jit
def kernel(inputs):
    mesh = plsc.VectorSubcoreMesh(core_axis_name="c", subcore_axis_name="s")
    run = pl.kernel(
        _body,
        out_type=jax.ShapeDtypeStruct((B, C, NBINS), jnp.float32),
        mesh=mesh,
        compiler_params=pltpu.CompilerParams(needs_layout_passes=False),
        scratch_types=[
            pltpu.VMEM((H16,), jnp.float32),
            pltpu.VMEM((NG * SG,), jnp.float32),
            pltpu.VMEM((HC, CPASS, W), jnp.float32),
            pltpu.VMEM((HC, CPASS, W), jnp.float32),
            pltpu.VMEM((C_PER_TILE, NBINS), jnp.float32),
            pltpu.SemaphoreType.DMA,
            pltpu.SemaphoreType.DMA,
        ],
    )
    xt = jnp.transpose(inputs, (0, 1, 3, 2))
    return jnp.transpose(run(xt), (0, 2, 1))


# speedup vs baseline: 2.0666x; 2.0666x over previous
import jax
import jax.numpy as jnp
from jax import lax
from jax.experimental import pallas as pl
from jax.experimental.pallas import tpu as pltpu
from jax.experimental.pallas import tpu_sc as plsc

B, H, W, C = 8, 224, 224, 96
NBINS = 256
NB_INT = NBINS + 1
TILES_PER_B = 4
C_PER_TILE = C // TILES_PER_B
CPASS = 8
NPASS = C_PER_TILE // CPASS
HC = 16
NCHUNK = H // HC
LANES = 16
WK = W // LANES
CSTRIDE = 258 * LANES
H16 = CPASS * CSTRIDE
NG = NBINS // LANES
SROW = 17
SG = LANES * SROW


def _body(x_hbm, out_hbm, h16_v, stg_v, buf0, buf1, outb_v, sem0, sem1):
    cid = lax.axis_index("c")
    sid = lax.axis_index("s")
    b = cid * TILES_PER_B + sid // TILES_PER_B
    q = sid % TILES_PER_B
    c0 = q * C_PER_TILE

    zeros = jnp.zeros((LANES,), jnp.float32)
    ones = jnp.ones((LANES,), jnp.float32)
    iota = lax.iota(jnp.int32, LANES)
    iota17 = iota * SROW

    @plsc.parallel_loop(0, H16 // LANES)
    def _(i):
        h16_v[pl.ds(i * LANES, LANES)] = zeros

    def chunk_compute(buf):
        @plsc.parallel_loop(0, CPASS * HC)
        def _(i):
            ci = i >> 4
            h = i & 15
            cb = iota + ci * CSTRIDE
            for k in range(WK):
                v = buf[h, ci, pl.ds(k * LANES, LANES)]
                idx = ((v * jnp.float32(NB_INT)).astype(jnp.int32) << 4) + cb
                plsc.addupdate_scatter(h16_v, [idx], ones)

    def start(p, n, buf, sem):
        pltpu.async_copy(
            x_hbm.at[b, pl.ds(n * HC, HC), pl.ds(c0 + p * CPASS, CPASS), :],
            buf, sem)

    def wait(buf, sem):
        pltpu.make_async_copy(
            x_hbm.at[0, pl.ds(0, HC), pl.ds(0, CPASS), :], buf, sem).wait()

    def pass_body(p, _):
        start(p, 0, buf0, sem0)

        def pair_body(k, _):
            start(p, 2 * k + 1, buf1, sem1)
            wait(buf0, sem0)
            chunk_compute(buf0)

            @pl.when(k < NCHUNK // 2 - 1)
            def _():
                start(p, 2 * k + 2, buf0, sem0)

            wait(buf1, sem1)
            chunk_compute(buf1)
            return 0

        lax.fori_loop(0, NCHUNK // 2, pair_body, 0)

        def fold_body(ci, _):
            cb2 = ci * CSTRIDE
            row = p * CPASS + ci

            @plsc.parallel_loop(0, NG, carry=(zeros,))
            def den_acc(g, dcar):
                sb = g * SG
                stg = stg_v.at[pl.ds(sb, SG)]
                for j2 in range(LANES):
                    o = cb2 + g * (LANES * LANES) + (j2 + 1) * LANES
                    v = h16_v[pl.ds(o, LANES)]
                    plsc.store_scatter(stg, [iota17 + j2], v)
                    h16_v[pl.ds(o, LANES)] = zeros
                acc = stg_v[pl.ds(sb, LANES)]
                for ll in range(1, LANES):
                    acc = acc + stg_v[pl.ds(sb + ll * SROW, LANES)]
                outb_v[row, pl.ds(g * LANES, LANES)] = acc
                return (dcar[0] + acc,)

            den = jnp.maximum(jnp.sum(den_acc[0]), jnp.float32(1e-7))
            inv = jnp.float32(1.0) / lax.broadcast_in_dim(den, (LANES,), ())
            for g in range(NG):
                outb_v[row, pl.ds(g * LANES, LANES)] = (
                    outb_v[row, pl.ds(g * LANES, LANES)] * inv)
            h16_v[pl.ds(cb2, LANES)] = zeros
            return 0

        lax.fori_loop(0, CPASS, fold_body, 0)
        return 0

    lax.fori_loop(0, NPASS, pass_body, 0)

    pltpu.sync_copy(outb_v, out_hbm.at[b, pl.ds(c0, C_PER_TILE), :])


@jax.jit
def kernel(inputs):
    mesh = plsc.VectorSubcoreMesh(core_axis_name="c", subcore_axis_name="s")
    run = pl.kernel(
        _body,
        out_type=jax.ShapeDtypeStruct((B, C, NBINS), jnp.float32),
        mesh=mesh,
        compiler_params=pltpu.CompilerParams(needs_layout_passes=False),
        scratch_types=[
            pltpu.VMEM((H16,), jnp.float32),
            pltpu.VMEM((NG * SG,), jnp.float32),
            pltpu.VMEM((HC, CPASS, W), jnp.float32),
            pltpu.VMEM((HC, CPASS, W), jnp.float32),
            pltpu.VMEM((C_PER_TILE, NBINS), jnp.float32),
            pltpu.SemaphoreType.DMA,
            pltpu.SemaphoreType.DMA,
        ],
    )
    xt = jnp.transpose(inputs, (0, 1, 3, 2))
    return jnp.transpose(run(xt), (0, 2, 1))


# speedup vs baseline: 2.0820x; 1.0075x over previous
import jax
import jax.numpy as jnp
from jax import lax
from jax.experimental import pallas as pl
from jax.experimental.pallas import tpu as pltpu
from jax.experimental.pallas import tpu_sc as plsc

B, H, W, C = 8, 224, 224, 96
NBINS = 256
NB_INT = NBINS + 1
TILES_PER_B = 4
C_PER_TILE = C // TILES_PER_B
CPASS = 8
NPASS = C_PER_TILE // CPASS
HC = 16
NCHUNK = H // HC
LANES = 16
WK = W // LANES
CSTRIDE = 258 * LANES
H16 = CPASS * CSTRIDE
NG = NBINS // LANES
SROW = 17
SG = LANES * SROW


def _body(x_hbm, out_hbm, h16_v, stg_v, buf0, buf1, outb_v, sem0, sem1):
    cid = lax.axis_index("c")
    sid = lax.axis_index("s")
    b = cid * TILES_PER_B + sid // TILES_PER_B
    q = sid % TILES_PER_B
    c0 = q * C_PER_TILE

    zeros = jnp.zeros((LANES,), jnp.float32)
    ones = jnp.ones((LANES,), jnp.float32)
    iota = lax.iota(jnp.int32, LANES)
    iota17 = iota * SROW

    @plsc.parallel_loop(0, H16 // LANES)
    def _(i):
        h16_v[pl.ds(i * LANES, LANES)] = zeros

    def chunk_compute(buf):
        @plsc.parallel_loop(0, CPASS * HC)
        def _(i):
            ci = i >> 4
            h = i & 15
            cb = iota + ci * CSTRIDE
            for k in range(WK):
                v = buf[h, ci, pl.ds(k * LANES, LANES)]
                idx = ((v * jnp.float32(NB_INT)).astype(jnp.int32) << 4) + cb
                plsc.addupdate_scatter(h16_v, [idx], ones)

    def start(p, n, buf, sem):
        pltpu.async_copy(
            x_hbm.at[b, pl.ds(n * HC, HC), pl.ds(c0 + p * CPASS, CPASS), :],
            buf, sem)

    def wait(buf, sem):
        pltpu.make_async_copy(
            x_hbm.at[0, pl.ds(0, HC), pl.ds(0, CPASS), :], buf, sem).wait()

    def pass_body(p, _):
        start(p, 0, buf0, sem0)

        def pair_body(k, _):
            start(p, 2 * k + 1, buf1, sem1)
            wait(buf0, sem0)
            chunk_compute(buf0)

            @pl.when(k < NCHUNK // 2 - 1)
            def _():
                start(p, 2 * k + 2, buf0, sem0)

            wait(buf1, sem1)
            chunk_compute(buf1)
            return 0

        lax.fori_loop(0, NCHUNK // 2, pair_body, 0)

        def fold_body(ci, _):
            cb2 = ci * CSTRIDE
            row = p * CPASS + ci

            row0 = h16_v[pl.ds(cb2, LANES)]
            h16_v[pl.ds(cb2, LANES)] = zeros
            den = jnp.maximum(jnp.float32(H * W) - jnp.sum(row0),
                              jnp.float32(1e-7))
            inv = jnp.float32(1.0) / lax.broadcast_in_dim(den, (LANES,), ())

            @plsc.parallel_loop(0, NG)
            def _(g):
                sb = g * SG
                stg = stg_v.at[pl.ds(sb, SG)]
                for j2 in range(LANES):
                    o = cb2 + g * (LANES * LANES) + (j2 + 1) * LANES
                    v = h16_v[pl.ds(o, LANES)]
                    plsc.store_scatter(stg, [iota17 + j2], v)
                    h16_v[pl.ds(o, LANES)] = zeros
                acc = stg_v[pl.ds(sb, LANES)]
                for ll in range(1, LANES):
                    acc = acc + stg_v[pl.ds(sb + ll * SROW, LANES)]
                outb_v[row, pl.ds(g * LANES, LANES)] = acc * inv
            return 0

        lax.fori_loop(0, CPASS, fold_body, 0)
        return 0

    lax.fori_loop(0, NPASS, pass_body, 0)

    pltpu.sync_copy(outb_v, out_hbm.at[b, pl.ds(c0, C_PER_TILE), :])


@jax.jit
def kernel(inputs):
    mesh = plsc.VectorSubcoreMesh(core_axis_name="c", subcore_axis_name="s")
    run = pl.kernel(
        _body,
        out_type=jax.ShapeDtypeStruct((B, C, NBINS), jnp.float32),
        mesh=mesh,
        compiler_params=pltpu.CompilerParams(needs_layout_passes=False),
        scratch_types=[
            pltpu.VMEM((H16,), jnp.float32),
            pltpu.VMEM((NG * SG,), jnp.float32),
            pltpu.VMEM((HC, CPASS, W), jnp.float32),
            pltpu.VMEM((HC, CPASS, W), jnp.float32),
            pltpu.VMEM((C_PER_TILE, NBINS), jnp.float32),
            pltpu.SemaphoreType.DMA,
            pltpu.SemaphoreType.DMA,
        ],
    )
    xt = jnp.transpose(inputs, (0, 1, 3, 2))
    return jnp.transpose(run(xt), (0, 2, 1))
